# baseline (speedup 1.0000x reference)
import functools

import jax
import jax.numpy as jnp
from jax import lax
from jax.experimental import pallas as pl
from jax.experimental.pallas import tpu as pltpu
from jax.experimental.pallas import tpu_sc as plsc

N = 10000
E = 320000
D = 128
L = 16
NC = 2
NS = 16
NW = NC * NS
NPAD = 10240

E_PER_TILE_DEG = E // NS
E_PER_TILE = E // NW
CHUNK = 80
NCHUNK = E_PER_TILE // CHUNK
ROWS_PER_TILE = NPAD // NS
ZROWS = 128

_mesh = plsc.VectorSubcoreMesh(core_axis_name="c", subcore_axis_name="s")
_sc_params = pltpu.CompilerParams(needs_layout_passes=False, use_tc_tiling_on_sc=False)


HROWS = 128


@functools.partial(
    pl.kernel,
    out_type=(jax.ShapeDtypeStruct((NPAD, D // 2), jnp.int32),
              jax.ShapeDtypeStruct((NPAD,), jnp.float32)),
    mesh=_mesh,
    scratch_types=[
        pltpu.VMEM((E_PER_TILE_DEG,), jnp.int32),
        pltpu.VMEM((NPAD,), jnp.float32),
        pltpu.VMEM((NPAD // NS,), jnp.float32),
        pltpu.VMEM((NPAD // NS,), jnp.float32),
        pltpu.VMEM((NPAD // NS,), jnp.float32),
        pltpu.VMEM((HROWS, D), jnp.float32),
        pltpu.VMEM((HROWS, D // 2), jnp.int32),
        pltpu.VMEM_SHARED((NS, NPAD), jnp.float32),
    ],
    compiler_params=_sc_params,
)
def _prep(src_hbm, dst_hbm, feat_hbm, hpk_hbm, cnt_hbm,
          idx_v, acc, tbuf, sbuf, nrm, fbuf, hbuf, shared):
    c = lax.axis_index("c")
    s = lax.axis_index("s")
    base = s * E_PER_TILE_DEG

    @pl.when(c == 0)
    def _():
        pltpu.sync_copy(src_hbm.at[pl.ds(base, E_PER_TILE_DEG)], idx_v)

    @pl.when(c == 1)
    def _():
        pltpu.sync_copy(dst_hbm.at[pl.ds(base, E_PER_TILE_DEG)], idx_v)

    zeros16 = jnp.zeros((L,), jnp.float32)
    ones16 = jnp.ones((L,), jnp.float32)

    def zero_body(i, _):
        acc[pl.ds(i * L, L)] = zeros16
        return 0

    lax.fori_loop(0, NPAD // L, zero_body, 0)

    def count_body(i, _):
        iv = idx_v[pl.ds(i * L, L)]
        plsc.addupdate_scatter(acc, [iv], ones16)
        return 0

    lax.fori_loop(0, E_PER_TILE_DEG // L, count_body, 0)

    pltpu.sync_copy(acc, shared.at[s])
    plsc.subcore_barrier()

    seg = NPAD // NS
    rbase = s * seg

    def zred(i, _):
        sbuf[pl.ds(i * L, L)] = zeros16
        return 0

    lax.fori_loop(0, seg // L, zred, 0)

    for t in range(NS):
        pltpu.sync_copy(shared.at[t, pl.ds(rbase, seg)], tbuf)

        def radd(i, _):
            sbuf[pl.ds(i * L, L)] = sbuf[pl.ds(i * L, L)] + tbuf[pl.ds(i * L, L)]
            return 0

        lax.fori_loop(0, seg // L, radd, 0)

    @pl.when(c == 1)
    def _():
        pltpu.sync_copy(sbuf, cnt_hbm.at[pl.ds(rbase, seg)])

    @pl.when(c == 0)
    def _():
        def nb(i, _):
            x = jnp.maximum(sbuf[pl.ds(i * L, L)], 1.0)
            iv = plsc.bitcast(x, jnp.int32)
            iv = jnp.int32(0x5F3759DF) - lax.shift_right_logical(iv, 1)
            y = plsc.bitcast(iv, jnp.float32)
            xh = 0.5 * x
            y = y * (1.5 - xh * y * y)
            y = y * (1.5 - xh * y * y)
            y = y * (1.5 - xh * y * y)
            nrm[pl.ds(i * L, L)] = y
            return 0

        lax.fori_loop(0, seg // L, nb, 0)

        for t in range(seg // HROWS):
            rb = rbase + t * HROWS
            pltpu.sync_copy(feat_hbm.at[pl.ds(rb, HROWS), :], fbuf)

            @plsc.parallel_loop(0, HROWS, 1, unroll=2)
            def _(r):
                rloc = t * HROWS + r
                gv = nrm[pl.ds((rloc // L) * L, L)]
                lane = jnp.full((L, 1), rloc % L, jnp.int32)
                bc = lax.gather(
                    gv, lane,
                    lax.GatherDimensionNumbers(
                        offset_dims=(), collapsed_slice_dims=(0,),
                        start_index_map=(0,)),
                    slice_sizes=(1,),
                    mode=lax.GatherScatterMode.PROMISE_IN_BOUNDS)
                for g in range(D // (2 * L)):
                    va = fbuf[r, pl.ds(2 * L * g, L)] * bc
                    vb = fbuf[r, pl.ds(2 * L * g + L, L)] * bc
                    pk = plsc.pack(va, vb, format=plsc.PackFormat.INTERLEAVED)
                    hbuf[r, pl.ds(L * g, L)] = plsc.bitcast(pk, jnp.int32)
            pltpu.sync_copy(hbuf, hpk_hbm.at[pl.ds(rb, HROWS), :])


@functools.partial(
    pl.kernel,
    out_type=jax.ShapeDtypeStruct((NC, NPAD, D), jnp.float32),
    mesh=_mesh,
    scratch_types=[
        pltpu.VMEM((2, CHUNK), jnp.int32),
        pltpu.VMEM((2, CHUNK), jnp.int32),
        pltpu.VMEM((2, CHUNK), jnp.int32),
        pltpu.VMEM((2, CHUNK), jnp.int32),
        pltpu.VMEM((CHUNK, D // 2), jnp.int32),
        pltpu.VMEM((CHUNK, D // 2), jnp.int32),
        pltpu.VMEM((CHUNK, D), jnp.float32),
        pltpu.VMEM((CHUNK, D), jnp.float32),
        pltpu.VMEM((CHUNK, D), jnp.float32),
        pltpu.VMEM_SHARED((NPAD, D), jnp.float32),
        pltpu.SemaphoreType.DMA,
        pltpu.SemaphoreType.DMA,
        pltpu.SemaphoreType.DMA,
        pltpu.SemaphoreType.DMA,
        pltpu.SemaphoreType.DMA,
        pltpu.SemaphoreType.DMA,
        pltpu.SemaphoreType.DMA,
        pltpu.SemaphoreType.DMA,
        pltpu.SemaphoreType.DMA,
    ],
    compiler_params=_sc_params,
)
def _aggregate(h_hbm, eidx_hbm, noise_hbm, out_hbm,
               ib0, ib1, ib2, ib3, rows_a, rows_b, nbuf_a, nbuf_b, mbuf, agg,
               si0, si1, si2, si3, sga, sgb, sna, snb, ssm):
    c = lax.axis_index("c")
    s = lax.axis_index("s")
    wid = c * NS + s
    ebase = wid * E_PER_TILE

    ibufs = [ib0, ib1, ib2, ib3]
    isems = [si0, si1, si2, si3]
    zeros16 = jnp.zeros((L,), jnp.float32)

    def zrow(r, _):
        def zcol(q, _):
            nbuf_a[r, pl.ds(q * L, L)] = zeros16
            return 0
        lax.fori_loop(0, D // L, zcol, 0, unroll=True)
        return 0

    lax.fori_loop(0, CHUNK, zrow, 0)

    abase = s * ROWS_PER_TILE
    for r in range(ROWS_PER_TILE // CHUNK):
        pltpu.sync_copy(nbuf_a, agg.at[pl.ds(abase + r * CHUNK, CHUNK), :])
    plsc.subcore_barrier()

    def fetch_idx(k, slot):
        pltpu.async_copy(eidx_hbm.at[wid, k], ibufs[slot], isems[slot])

    def wait_idx(k, slot):
        pltpu.make_async_copy(eidx_hbm.at[wid, k], ibufs[slot], isems[slot]).wait()

    H = CHUNK // 2

    def fetch_data(k, slot, rows_v, nbuf_v, sg, sn):
        ib = ibufs[slot]
        pltpu.async_copy(h_hbm.at[ib.at[0, pl.ds(0, H)]], rows_v.at[pl.ds(0, H), :], sg)
        pltpu.async_copy(h_hbm.at[ib.at[0, pl.ds(H, H)]], rows_v.at[pl.ds(H, H), :], sg)
        b = ebase + k * CHUNK
        pltpu.async_copy(noise_hbm.at[pl.ds(b, H), :], nbuf_v.at[pl.ds(0, H), :], sn)
        pltpu.async_copy(
            noise_hbm.at[pl.ds(b + H, H), :], nbuf_v.at[pl.ds(H, H), :], sn)

    def wait_data(k, slot, rows_v, nbuf_v, sg, sn):
        ib = ibufs[slot]
        pltpu.make_async_copy(
            h_hbm.at[ib.at[0, pl.ds(0, H)]], rows_v.at[pl.ds(0, H), :], sg).wait()
        pltpu.make_async_copy(
            h_hbm.at[ib.at[0, pl.ds(H, H)]], rows_v.at[pl.ds(H, H), :], sg).wait()
        b = ebase + k * CHUNK
        pltpu.make_async_copy(
            noise_hbm.at[pl.ds(b, H), :], nbuf_v.at[pl.ds(0, H), :], sn).wait()
        pltpu.make_async_copy(
            noise_hbm.at[pl.ds(b + H, H), :], nbuf_v.at[pl.ds(H, H), :], sn).wait()

    def compute(rows_v, nbuf_v):
        @plsc.parallel_loop(0, CHUNK, 1, unroll=2)
        def _(e):
            for g in range(D // (2 * L)):
                hw = rows_v[e, pl.ds(L * g, L)]
                hb = plsc.bitcast(hw, jnp.bfloat16)
                va, vb = plsc.unpack(hb, format=plsc.PackFormat.INTERLEAVED)
                wa = nbuf_v[e, pl.ds(2 * L * g, L)]
                wb = nbuf_v[e, pl.ds(2 * L * g + L, L)]
                mbuf[e, pl.ds(2 * L * g, L)] = va + va * wa
                mbuf[e, pl.ds(2 * L * g + L, L)] = vb + vb * wb

    def scatter(slot):
        pltpu.async_copy(mbuf, agg.at[ibufs[slot].at[1]], ssm, add=True)

    def wait_scatter(slot):
        pltpu.make_async_copy(mbuf, agg.at[ibufs[slot].at[1]], ssm).wait()

    pltpu.sync_copy(eidx_hbm.at[wid, 0], ib0)
    pltpu.sync_copy(eidx_hbm.at[wid, 1], ib1)
    fetch_data(0, 0, rows_a, nbuf_a, sga, sna)
    fetch_data(1, 1, rows_b, nbuf_b, sgb, snb)

    bufs = ((rows_a, nbuf_a, sga, sna), (rows_b, nbuf_b, sgb, snb))

    def step(k, slot, fetch_next, drain_prev=True):
        rows_v, nbuf_v, sg, sn = bufs[slot % 2]
        nslot = (slot + 2) % 4
        if fetch_next:
            fetch_idx(k + 2, nslot)
        wait_data(k, slot, rows_v, nbuf_v, sg, sn)
        if drain_prev:
            wait_scatter((slot + 3) % 4)
        compute(rows_v, nbuf_v)
        scatter(slot)
        if fetch_next:
            wait_idx(k + 2, nslot)
            fetch_data(k + 2, nslot, rows_v, nbuf_v, sg, sn)

    step(0, 0, True, drain_prev=False)
    step(1, 1, True)
    step(2, 2, True)
    step(3, 3, True)

    def quad(j, _):
        k0 = j * 4
        for d in range(4):
            step(k0 + d, d, True)
        return 0

    lax.fori_loop(1, (NCHUNK - 5) // 4, quad, 0)

    base = NCHUNK - 5
    step(base + 0, 0, True)
    step(base + 1, 1, True)
    step(base + 2, 2, True)
    step(base + 3, 3, False)
    step(base + 4, 0, False)
    wait_scatter((base + 4) % 4)

    plsc.subcore_barrier()
    for r in range(ROWS_PER_TILE // CHUNK):
        rb = abase + r * CHUNK
        pltpu.sync_copy(agg.at[pl.ds(rb, CHUNK), :], out_hbm.at[c, pl.ds(rb, CHUNK), :])


def _finish_body(p0, p1, cnt, w, bb, o):
    agg = p0[...] + p1[...]
    scale = lax.rsqrt(jnp.maximum(cnt[...], 1.0))
    o[...] = jnp.dot(agg * scale, w[...], preferred_element_type=jnp.float32) + bb[...]


_FIN_BLOCK = 2000


def _finish(p0, p1, cnt2d, W, b2d):
    return pl.pallas_call(
        _finish_body,
        out_shape=jax.ShapeDtypeStruct((N, D), jnp.float32),
        grid=(N // _FIN_BLOCK,),
        in_specs=[
            pl.BlockSpec((_FIN_BLOCK, D), lambda i: (i, 0)),
            pl.BlockSpec((_FIN_BLOCK, D), lambda i: (i, 0)),
            pl.BlockSpec((_FIN_BLOCK, 1), lambda i: (i, 0)),
            pl.BlockSpec((D, D), lambda i: (0, 0)),
            pl.BlockSpec((1, D), lambda i: (0, 0)),
        ],
        out_specs=pl.BlockSpec((_FIN_BLOCK, D), lambda i: (i, 0)),
    )(p0, p1, cnt2d, W, b2d)


def kernel(feat, edge_index, edge_noise, W, b):
    src = edge_index[0]
    dst = edge_index[1]
    fpad = jnp.pad(feat, ((0, NPAD - N), (0, 0)))
    h_pk, cnt = _prep(src, dst, fpad)
    eidx = edge_index.reshape(2, NW, NCHUNK, CHUNK).transpose(1, 2, 0, 3)
    parts = _aggregate(h_pk, eidx, edge_noise)
    cnt_dst = cnt[:N].reshape(N, 1)
    return _finish(parts[0, :N], parts[1, :N], cnt_dst, W, b.reshape(1, D))


# speedup vs baseline: 1.0048x; 1.0048x over previous
import functools

import jax
import jax.numpy as jnp
from jax import lax
from jax.experimental import pallas as pl
from jax.experimental.pallas import tpu as pltpu
from jax.experimental.pallas import tpu_sc as plsc

N = 10000
E = 320000
D = 128
L = 16
NC = 2
NS = 16
NW = NC * NS
NPAD = 10240

E_PER_TILE_DEG = E // NS
E_PER_TILE = E // NW
CHUNK = 80
NCHUNK = E_PER_TILE // CHUNK
ROWS_PER_TILE = NPAD // NS
ZROWS = 128

_mesh = plsc.VectorSubcoreMesh(core_axis_name="c", subcore_axis_name="s")
_sc_params = pltpu.CompilerParams(needs_layout_passes=False, use_tc_tiling_on_sc=False)


HROWS = 128


@functools.partial(
    pl.kernel,
    out_type=(jax.ShapeDtypeStruct((NPAD, D // 2), jnp.int32),
              jax.ShapeDtypeStruct((NPAD,), jnp.float32)),
    mesh=_mesh,
    scratch_types=[
        pltpu.VMEM((E_PER_TILE_DEG,), jnp.int32),
        pltpu.VMEM((NPAD,), jnp.float32),
        pltpu.VMEM((NS, NPAD // NS), jnp.float32),
        pltpu.VMEM((NPAD // NS,), jnp.float32),
        pltpu.VMEM((NPAD // NS,), jnp.float32),
        pltpu.VMEM((HROWS, D), jnp.float32),
        pltpu.VMEM((HROWS, D // 2), jnp.int32),
        pltpu.VMEM_SHARED((NS, NPAD), jnp.float32),
    ],
    compiler_params=_sc_params,
)
def _prep(src_hbm, dst_hbm, feat_hbm, hpk_hbm, cnt_hbm,
          idx_v, acc, tbuf, sbuf, nrm, fbuf, hbuf, shared):
    c = lax.axis_index("c")
    s = lax.axis_index("s")
    base = s * E_PER_TILE_DEG

    @pl.when(c == 0)
    def _():
        pltpu.sync_copy(src_hbm.at[pl.ds(base, E_PER_TILE_DEG)], idx_v)

    @pl.when(c == 1)
    def _():
        pltpu.sync_copy(dst_hbm.at[pl.ds(base, E_PER_TILE_DEG)], idx_v)

    zeros16 = jnp.zeros((L,), jnp.float32)
    ones16 = jnp.ones((L,), jnp.float32)

    def zero_body(i, _):
        acc[pl.ds(i * L, L)] = zeros16
        return 0

    lax.fori_loop(0, NPAD // L, zero_body, 0)

    def count_body(i, _):
        iv = idx_v[pl.ds(i * L, L)]
        plsc.addupdate_scatter(acc, [iv], ones16)
        return 0

    lax.fori_loop(0, E_PER_TILE_DEG // L, count_body, 0)

    pltpu.sync_copy(acc, shared.at[s])
    plsc.subcore_barrier()

    seg = NPAD // NS
    rbase = s * seg

    def zred(i, _):
        sbuf[pl.ds(i * L, L)] = zeros16
        return 0

    lax.fori_loop(0, seg // L, zred, 0)

    pltpu.sync_copy(shared.at[:, pl.ds(rbase, seg)], tbuf)
    for t in range(NS):
        def radd(i, _):
            sbuf[pl.ds(i * L, L)] = sbuf[pl.ds(i * L, L)] + tbuf[t, pl.ds(i * L, L)]
            return 0

        lax.fori_loop(0, seg // L, radd, 0)

    @pl.when(c == 1)
    def _():
        pltpu.sync_copy(sbuf, cnt_hbm.at[pl.ds(rbase, seg)])

    @pl.when(c == 0)
    def _():
        def nb(i, _):
            x = jnp.maximum(sbuf[pl.ds(i * L, L)], 1.0)
            iv = plsc.bitcast(x, jnp.int32)
            iv = jnp.int32(0x5F3759DF) - lax.shift_right_logical(iv, 1)
            y = plsc.bitcast(iv, jnp.float32)
            xh = 0.5 * x
            y = y * (1.5 - xh * y * y)
            y = y * (1.5 - xh * y * y)
            y = y * (1.5 - xh * y * y)
            nrm[pl.ds(i * L, L)] = y
            return 0

        lax.fori_loop(0, seg // L, nb, 0)

        for t in range(seg // HROWS):
            rb = rbase + t * HROWS
            pltpu.sync_copy(feat_hbm.at[pl.ds(rb, HROWS), :], fbuf)

            @plsc.parallel_loop(0, HROWS, 1, unroll=2)
            def _(r):
                rloc = t * HROWS + r
                gv = nrm[pl.ds((rloc // L) * L, L)]
                lane = jnp.full((L, 1), rloc % L, jnp.int32)
                bc = lax.gather(
                    gv, lane,
                    lax.GatherDimensionNumbers(
                        offset_dims=(), collapsed_slice_dims=(0,),
                        start_index_map=(0,)),
                    slice_sizes=(1,),
                    mode=lax.GatherScatterMode.PROMISE_IN_BOUNDS)
                for g in range(D // (2 * L)):
                    va = fbuf[r, pl.ds(2 * L * g, L)] * bc
                    vb = fbuf[r, pl.ds(2 * L * g + L, L)] * bc
                    pk = plsc.pack(va, vb, format=plsc.PackFormat.INTERLEAVED)
                    hbuf[r, pl.ds(L * g, L)] = plsc.bitcast(pk, jnp.int32)
            pltpu.sync_copy(hbuf, hpk_hbm.at[pl.ds(rb, HROWS), :])


@functools.partial(
    pl.kernel,
    out_type=jax.ShapeDtypeStruct((NC, NPAD, D), jnp.float32),
    mesh=_mesh,
    scratch_types=[
        pltpu.VMEM((2, CHUNK), jnp.int32),
        pltpu.VMEM((2, CHUNK), jnp.int32),
        pltpu.VMEM((2, CHUNK), jnp.int32),
        pltpu.VMEM((2, CHUNK), jnp.int32),
        pltpu.VMEM((CHUNK, D // 2), jnp.int32),
        pltpu.VMEM((CHUNK, D // 2), jnp.int32),
        pltpu.VMEM((CHUNK, D), jnp.float32),
        pltpu.VMEM((CHUNK, D), jnp.float32),
        pltpu.VMEM((CHUNK, D), jnp.float32),
        pltpu.VMEM_SHARED((NPAD, D), jnp.float32),
        pltpu.SemaphoreType.DMA,
        pltpu.SemaphoreType.DMA,
        pltpu.SemaphoreType.DMA,
        pltpu.SemaphoreType.DMA,
        pltpu.SemaphoreType.DMA,
        pltpu.SemaphoreType.DMA,
        pltpu.SemaphoreType.DMA,
        pltpu.SemaphoreType.DMA,
        pltpu.SemaphoreType.DMA,
    ],
    compiler_params=_sc_params,
)
def _aggregate(h_hbm, eidx_hbm, noise_hbm, out_hbm,
               ib0, ib1, ib2, ib3, rows_a, rows_b, nbuf_a, nbuf_b, mbuf, agg,
               si0, si1, si2, si3, sga, sgb, sna, snb, ssm):
    c = lax.axis_index("c")
    s = lax.axis_index("s")
    wid = c * NS + s
    ebase = wid * E_PER_TILE

    ibufs = [ib0, ib1, ib2, ib3]
    isems = [si0, si1, si2, si3]
    zeros16 = jnp.zeros((L,), jnp.float32)

    def zrow(r, _):
        def zcol(q, _):
            nbuf_a[r, pl.ds(q * L, L)] = zeros16
            return 0
        lax.fori_loop(0, D // L, zcol, 0, unroll=True)
        return 0

    lax.fori_loop(0, CHUNK, zrow, 0)

    abase = s * ROWS_PER_TILE
    for r in range(ROWS_PER_TILE // CHUNK):
        pltpu.sync_copy(nbuf_a, agg.at[pl.ds(abase + r * CHUNK, CHUNK), :])
    plsc.subcore_barrier()

    def fetch_idx(k, slot):
        pltpu.async_copy(eidx_hbm.at[wid, k], ibufs[slot], isems[slot])

    def wait_idx(k, slot):
        pltpu.make_async_copy(eidx_hbm.at[wid, k], ibufs[slot], isems[slot]).wait()

    H = CHUNK // 2

    def fetch_data(k, slot, rows_v, nbuf_v, sg, sn):
        pltpu.async_copy(h_hbm.at[ibufs[slot].at[0]], rows_v, sg)
        b = ebase + k * CHUNK
        pltpu.async_copy(noise_hbm.at[pl.ds(b, H), :], nbuf_v.at[pl.ds(0, H), :], sn)
        pltpu.async_copy(
            noise_hbm.at[pl.ds(b + H, H), :], nbuf_v.at[pl.ds(H, H), :], sn)

    def wait_data(k, slot, rows_v, nbuf_v, sg, sn):
        pltpu.make_async_copy(h_hbm.at[ibufs[slot].at[0]], rows_v, sg).wait()
        b = ebase + k * CHUNK
        pltpu.make_async_copy(
            noise_hbm.at[pl.ds(b, H), :], nbuf_v.at[pl.ds(0, H), :], sn).wait()
        pltpu.make_async_copy(
            noise_hbm.at[pl.ds(b + H, H), :], nbuf_v.at[pl.ds(H, H), :], sn).wait()

    def compute(rows_v, nbuf_v):
        @plsc.parallel_loop(0, CHUNK, 1, unroll=2)
        def _(e):
            for g in range(D // (2 * L)):
                hw = rows_v[e, pl.ds(L * g, L)]
                hb = plsc.bitcast(hw, jnp.bfloat16)
                va, vb = plsc.unpack(hb, format=plsc.PackFormat.INTERLEAVED)
                wa = nbuf_v[e, pl.ds(2 * L * g, L)]
                wb = nbuf_v[e, pl.ds(2 * L * g + L, L)]
                mbuf[e, pl.ds(2 * L * g, L)] = va + va * wa
                mbuf[e, pl.ds(2 * L * g + L, L)] = vb + vb * wb

    def scatter(slot):
        pltpu.async_copy(mbuf, agg.at[ibufs[slot].at[1]], ssm, add=True)

    def wait_scatter(slot):
        pltpu.make_async_copy(mbuf, agg.at[ibufs[slot].at[1]], ssm).wait()

    pltpu.sync_copy(eidx_hbm.at[wid, 0], ib0)
    pltpu.sync_copy(eidx_hbm.at[wid, 1], ib1)
    fetch_data(0, 0, rows_a, nbuf_a, sga, sna)
    fetch_data(1, 1, rows_b, nbuf_b, sgb, snb)

    bufs = ((rows_a, nbuf_a, sga, sna), (rows_b, nbuf_b, sgb, snb))

    def step(k, slot, fetch_next, drain_prev=True):
        rows_v, nbuf_v, sg, sn = bufs[slot % 2]
        nslot = (slot + 2) % 4
        if fetch_next:
            fetch_idx(k + 2, nslot)
        wait_data(k, slot, rows_v, nbuf_v, sg, sn)
        if drain_prev:
            wait_scatter((slot + 3) % 4)
        compute(rows_v, nbuf_v)
        scatter(slot)
        if fetch_next:
            wait_idx(k + 2, nslot)
            fetch_data(k + 2, nslot, rows_v, nbuf_v, sg, sn)

    step(0, 0, True, drain_prev=False)
    step(1, 1, True)
    step(2, 2, True)
    step(3, 3, True)

    def quad(j, _):
        k0 = j * 4
        for d in range(4):
            step(k0 + d, d, True)
        return 0

    lax.fori_loop(1, (NCHUNK - 5) // 4, quad, 0)

    base = NCHUNK - 5
    step(base + 0, 0, True)
    step(base + 1, 1, True)
    step(base + 2, 2, True)
    step(base + 3, 3, False)
    step(base + 4, 0, False)
    wait_scatter((base + 4) % 4)

    plsc.subcore_barrier()
    for r in range(ROWS_PER_TILE // CHUNK):
        rb = abase + r * CHUNK
        pltpu.sync_copy(agg.at[pl.ds(rb, CHUNK), :], out_hbm.at[c, pl.ds(rb, CHUNK), :])


def _finish_body(p0, p1, cnt, w, bb, o):
    agg = p0[...] + p1[...]
    scale = lax.rsqrt(jnp.maximum(cnt[...], 1.0))
    o[...] = jnp.dot(agg * scale, w[...], preferred_element_type=jnp.float32) + bb[...]


_FIN_BLOCK = 2000


def _finish(p0, p1, cnt2d, W, b2d):
    return pl.pallas_call(
        _finish_body,
        out_shape=jax.ShapeDtypeStruct((N, D), jnp.float32),
        grid=(N // _FIN_BLOCK,),
        in_specs=[
            pl.BlockSpec((_FIN_BLOCK, D), lambda i: (i, 0)),
            pl.BlockSpec((_FIN_BLOCK, D), lambda i: (i, 0)),
            pl.BlockSpec((_FIN_BLOCK, 1), lambda i: (i, 0)),
            pl.BlockSpec((D, D), lambda i: (0, 0)),
            pl.BlockSpec((1, D), lambda i: (0, 0)),
        ],
        out_specs=pl.BlockSpec((_FIN_BLOCK, D), lambda i: (i, 0)),
    )(p0, p1, cnt2d, W, b2d)


def kernel(feat, edge_index, edge_noise, W, b):
    src = edge_index[0]
    dst = edge_index[1]
    fpad = jnp.pad(feat, ((0, NPAD - N), (0, 0)))
    h_pk, cnt = _prep(src, dst, fpad)
    eidx = edge_index.reshape(2, NW, NCHUNK, CHUNK).transpose(1, 2, 0, 3)
    parts = _aggregate(h_pk, eidx, edge_noise)
    cnt_dst = cnt[:N].reshape(N, 1)
    return _finish(parts[0, :N], parts[1, :N], cnt_dst, W, b.reshape(1, D))


# speedup vs baseline: 1.0056x; 1.0008x over previous
import functools

import jax
import jax.numpy as jnp
from jax import lax
from jax.experimental import pallas as pl
from jax.experimental.pallas import tpu as pltpu
from jax.experimental.pallas import tpu_sc as plsc

N = 10000
E = 320000
D = 128
L = 16
NC = 2
NS = 16
NW = NC * NS
NPAD = 10240

E_PER_TILE_DEG = E // NS
E_PER_TILE = E // NW
CHUNK = 80
NCHUNK = E_PER_TILE // CHUNK
ROWS_PER_TILE = NPAD // NS

_mesh = plsc.VectorSubcoreMesh(core_axis_name="c", subcore_axis_name="s")
_sc_params = pltpu.CompilerParams(needs_layout_passes=False, use_tc_tiling_on_sc=False)


HROWS = 128


@functools.partial(
    pl.kernel,
    out_type=(jax.ShapeDtypeStruct((NPAD, D // 2), jnp.int32),
              jax.ShapeDtypeStruct((NPAD,), jnp.float32)),
    mesh=_mesh,
    scratch_types=[
        pltpu.VMEM((E_PER_TILE_DEG,), jnp.int32),
        pltpu.VMEM((NPAD,), jnp.float32),
        pltpu.VMEM((NS, NPAD // NS), jnp.float32),
        pltpu.VMEM((NPAD // NS,), jnp.float32),
        pltpu.VMEM((NPAD // NS,), jnp.float32),
        pltpu.VMEM((HROWS, D), jnp.float32),
        pltpu.VMEM((HROWS, D // 2), jnp.int32),
        pltpu.VMEM_SHARED((NS, NPAD), jnp.float32),
    ],
    compiler_params=_sc_params,
)
def _prep(src_hbm, dst_hbm, feat_hbm, hpk_hbm, cnt_hbm,
          idx_v, acc, tbuf, sbuf, nrm, fbuf, hbuf, shared):
    c = lax.axis_index("c")
    s = lax.axis_index("s")
    base = s * E_PER_TILE_DEG

    @pl.when(c == 0)
    def _():
        pltpu.sync_copy(src_hbm.at[pl.ds(base, E_PER_TILE_DEG)], idx_v)

    @pl.when(c == 1)
    def _():
        pltpu.sync_copy(dst_hbm.at[pl.ds(base, E_PER_TILE_DEG)], idx_v)

    zeros16 = jnp.zeros((L,), jnp.float32)
    ones16 = jnp.ones((L,), jnp.float32)

    def zero_body(i, _):
        acc[pl.ds(i * L, L)] = zeros16
        return 0

    lax.fori_loop(0, NPAD // L, zero_body, 0)

    def count_body(i, _):
        iv = idx_v[pl.ds(i * L, L)]
        plsc.addupdate_scatter(acc, [iv], ones16)
        return 0

    lax.fori_loop(0, E_PER_TILE_DEG // L, count_body, 0)

    pltpu.sync_copy(acc, shared.at[s])
    plsc.subcore_barrier()

    seg = NPAD // NS
    rbase = s * seg

    def zred(i, _):
        sbuf[pl.ds(i * L, L)] = zeros16
        return 0

    lax.fori_loop(0, seg // L, zred, 0)

    pltpu.sync_copy(shared.at[:, pl.ds(rbase, seg)], tbuf)
    for t in range(NS):
        def radd(i, _):
            sbuf[pl.ds(i * L, L)] = sbuf[pl.ds(i * L, L)] + tbuf[t, pl.ds(i * L, L)]
            return 0

        lax.fori_loop(0, seg // L, radd, 0)

    @pl.when(c == 1)
    def _():
        pltpu.sync_copy(sbuf, cnt_hbm.at[pl.ds(rbase, seg)])

    @pl.when(c == 0)
    def _():
        def nb(i, _):
            x = jnp.maximum(sbuf[pl.ds(i * L, L)], 1.0)
            iv = plsc.bitcast(x, jnp.int32)
            iv = jnp.int32(0x5F3759DF) - lax.shift_right_logical(iv, 1)
            y = plsc.bitcast(iv, jnp.float32)
            xh = 0.5 * x
            y = y * (1.5 - xh * y * y)
            y = y * (1.5 - xh * y * y)
            y = y * (1.5 - xh * y * y)
            nrm[pl.ds(i * L, L)] = y
            return 0

        lax.fori_loop(0, seg // L, nb, 0)

        for t in range(seg // HROWS):
            rb = rbase + t * HROWS
            pltpu.sync_copy(feat_hbm.at[pl.ds(rb, HROWS), :], fbuf)

            @plsc.parallel_loop(0, HROWS, 1, unroll=2)
            def _(r):
                rloc = t * HROWS + r
                gv = nrm[pl.ds((rloc // L) * L, L)]
                lane = jnp.full((L, 1), rloc % L, jnp.int32)
                bc = lax.gather(
                    gv, lane,
                    lax.GatherDimensionNumbers(
                        offset_dims=(), collapsed_slice_dims=(0,),
                        start_index_map=(0,)),
                    slice_sizes=(1,),
                    mode=lax.GatherScatterMode.PROMISE_IN_BOUNDS)
                for g in range(D // (2 * L)):
                    va = fbuf[r, pl.ds(2 * L * g, L)] * bc
                    vb = fbuf[r, pl.ds(2 * L * g + L, L)] * bc
                    pk = plsc.pack(va, vb, format=plsc.PackFormat.INTERLEAVED)
                    hbuf[r, pl.ds(L * g, L)] = plsc.bitcast(pk, jnp.int32)
            pltpu.sync_copy(hbuf, hpk_hbm.at[pl.ds(rb, HROWS), :])


@functools.partial(
    pl.kernel,
    out_type=jax.ShapeDtypeStruct((NC, NPAD, D), jnp.float32),
    mesh=_mesh,
    scratch_types=[
        pltpu.VMEM((2, CHUNK), jnp.int32),
        pltpu.VMEM((2, CHUNK), jnp.int32),
        pltpu.VMEM((2, CHUNK), jnp.int32),
        pltpu.VMEM((2, CHUNK), jnp.int32),
        pltpu.VMEM((CHUNK, D // 2), jnp.int32),
        pltpu.VMEM((CHUNK, D // 2), jnp.int32),
        pltpu.VMEM((CHUNK, D), jnp.float32),
        pltpu.VMEM((CHUNK, D), jnp.float32),
        pltpu.VMEM((CHUNK, D), jnp.float32),
        pltpu.VMEM_SHARED((NPAD, D), jnp.float32),
        pltpu.SemaphoreType.DMA,
        pltpu.SemaphoreType.DMA,
        pltpu.SemaphoreType.DMA,
        pltpu.SemaphoreType.DMA,
        pltpu.SemaphoreType.DMA,
        pltpu.SemaphoreType.DMA,
        pltpu.SemaphoreType.DMA,
        pltpu.SemaphoreType.DMA,
        pltpu.SemaphoreType.DMA,
    ],
    compiler_params=_sc_params,
)
def _aggregate(h_hbm, eidx_hbm, noise_hbm, out_hbm,
               ib0, ib1, ib2, ib3, rows_a, rows_b, nbuf_a, nbuf_b, mbuf, agg,
               si0, si1, si2, si3, sga, sgb, sna, snb, ssm):
    c = lax.axis_index("c")
    s = lax.axis_index("s")
    wid = c * NS + s
    ebase = wid * E_PER_TILE

    ibufs = [ib0, ib1, ib2, ib3]
    isems = [si0, si1, si2, si3]
    zeros16 = jnp.zeros((L,), jnp.float32)

    def zrow(r, _):
        def zcol(q, _):
            nbuf_a[r, pl.ds(q * L, L)] = zeros16
            return 0
        lax.fori_loop(0, D // L, zcol, 0, unroll=True)
        return 0

    lax.fori_loop(0, CHUNK, zrow, 0)

    abase = s * ROWS_PER_TILE
    for r in range(ROWS_PER_TILE // CHUNK):
        pltpu.sync_copy(nbuf_a, agg.at[pl.ds(abase + r * CHUNK, CHUNK), :])
    plsc.subcore_barrier()

    def fetch_idx(k, slot):
        pltpu.async_copy(eidx_hbm.at[wid, k], ibufs[slot], isems[slot])

    def wait_idx(k, slot):
        pltpu.make_async_copy(eidx_hbm.at[wid, k], ibufs[slot], isems[slot]).wait()

    H = CHUNK // 2

    def fetch_data(k, slot, rows_v, nbuf_v, sg, sn):
        pltpu.async_copy(h_hbm.at[ibufs[slot].at[0]], rows_v, sg)
        b = ebase + k * CHUNK
        pltpu.async_copy(noise_hbm.at[pl.ds(b, H), :], nbuf_v.at[pl.ds(0, H), :], sn)
        pltpu.async_copy(
            noise_hbm.at[pl.ds(b + H, H), :], nbuf_v.at[pl.ds(H, H), :], sn)

    def wait_data(k, slot, rows_v, nbuf_v, sg, sn):
        pltpu.make_async_copy(h_hbm.at[ibufs[slot].at[0]], rows_v, sg).wait()
        b = ebase + k * CHUNK
        pltpu.make_async_copy(
            noise_hbm.at[pl.ds(b, H), :], nbuf_v.at[pl.ds(0, H), :], sn).wait()
        pltpu.make_async_copy(
            noise_hbm.at[pl.ds(b + H, H), :], nbuf_v.at[pl.ds(H, H), :], sn).wait()

    def compute(rows_v, nbuf_v):
        @plsc.parallel_loop(0, CHUNK, 1, unroll=2)
        def _(e):
            for g in range(D // (2 * L)):
                hw = rows_v[e, pl.ds(L * g, L)]
                hb = plsc.bitcast(hw, jnp.bfloat16)
                va, vb = plsc.unpack(hb, format=plsc.PackFormat.INTERLEAVED)
                wa = nbuf_v[e, pl.ds(2 * L * g, L)]
                wb = nbuf_v[e, pl.ds(2 * L * g + L, L)]
                mbuf[e, pl.ds(2 * L * g, L)] = va + va * wa
                mbuf[e, pl.ds(2 * L * g + L, L)] = vb + vb * wb

    def scatter(slot):
        pltpu.async_copy(mbuf, agg.at[ibufs[slot].at[1]], ssm, add=True)

    def wait_scatter(slot):
        pltpu.make_async_copy(mbuf, agg.at[ibufs[slot].at[1]], ssm).wait()

    pltpu.sync_copy(eidx_hbm.at[wid, 0], ib0)
    pltpu.sync_copy(eidx_hbm.at[wid, 1], ib1)
    fetch_data(0, 0, rows_a, nbuf_a, sga, sna)
    fetch_data(1, 1, rows_b, nbuf_b, sgb, snb)

    bufs = ((rows_a, nbuf_a, sga, sna), (rows_b, nbuf_b, sgb, snb))

    def step(k, slot, fetch_next, drain_prev=True):
        rows_v, nbuf_v, sg, sn = bufs[slot % 2]
        nslot = (slot + 2) % 4
        if fetch_next:
            fetch_idx(k + 2, nslot)
        wait_data(k, slot, rows_v, nbuf_v, sg, sn)
        if drain_prev:
            wait_scatter((slot + 3) % 4)
        compute(rows_v, nbuf_v)
        scatter(slot)
        if fetch_next:
            wait_idx(k + 2, nslot)
            fetch_data(k + 2, nslot, rows_v, nbuf_v, sg, sn)

    step(0, 0, True, drain_prev=False)
    step(1, 1, True)
    step(2, 2, True)
    step(3, 3, True)

    def quad(j, _):
        k0 = j * 4
        for d in range(4):
            step(k0 + d, d, True)
        return 0

    lax.fori_loop(1, (NCHUNK - 5) // 4, quad, 0)

    base = NCHUNK - 5
    step(base + 0, 0, True)
    step(base + 1, 1, True)
    step(base + 2, 2, True)
    step(base + 3, 3, False)
    step(base + 4, 0, False)
    wait_scatter((base + 4) % 4)

    plsc.subcore_barrier()
    for r in range(ROWS_PER_TILE // CHUNK):
        rb = abase + r * CHUNK
        pltpu.sync_copy(agg.at[pl.ds(rb, CHUNK), :], out_hbm.at[c, pl.ds(rb, CHUNK), :])


def _finish_body(p0, p1, cnt, w, bb, o):
    agg = p0[...] + p1[...]
    scale = lax.rsqrt(jnp.maximum(cnt[...], 1.0))
    o[...] = jnp.dot(agg * scale, w[...], preferred_element_type=jnp.float32) + bb[...]


_FIN_BLOCK = 2000


def _finish(p0, p1, cnt2d, W, b2d):
    return pl.pallas_call(
        _finish_body,
        out_shape=jax.ShapeDtypeStruct((N, D), jnp.float32),
        grid=(N // _FIN_BLOCK,),
        in_specs=[
            pl.BlockSpec((_FIN_BLOCK, D), lambda i: (i, 0)),
            pl.BlockSpec((_FIN_BLOCK, D), lambda i: (i, 0)),
            pl.BlockSpec((_FIN_BLOCK, 1), lambda i: (i, 0)),
            pl.BlockSpec((D, D), lambda i: (0, 0)),
            pl.BlockSpec((1, D), lambda i: (0, 0)),
        ],
        out_specs=pl.BlockSpec((_FIN_BLOCK, D), lambda i: (i, 0)),
    )(p0, p1, cnt2d, W, b2d)


def kernel(feat, edge_index, edge_noise, W, b):
    src = edge_index[0]
    dst = edge_index[1]
    fpad = jnp.pad(feat, ((0, NPAD - N), (0, 0)))
    h_pk, cnt = _prep(src, dst, fpad)
    eidx = edge_index.reshape(2, NW, NCHUNK, CHUNK).transpose(1, 2, 0, 3)
    parts = _aggregate(h_pk, eidx, edge_noise)
    cnt_dst = cnt[:N].reshape(N, 1)
    return _finish(parts[0, :N], parts[1, :N], cnt_dst, W, b.reshape(1, D))
